# baseline (speedup 1.0000x reference)
import functools

import jax
import jax.numpy as jnp
from jax import lax
from jax.experimental import pallas as pl
from jax.experimental.pallas import tpu as pltpu
from jax.experimental.pallas import tpu_sc as plsc

_NC = 2
_NS = 16
_NW = _NC * _NS
_L = 16
_CH = 8


def _worker_id():
    return lax.axis_index("s") * _NC + lax.axis_index("c")


def _chunk_sum(buf, coef_a, coef_b, acc):
    kdim = buf.shape[1]

    def row_body(r, acc):
        def vec_body(v, acc):
            p = buf[r, pl.ds(v * _L, _L)]
            y = p * coef_a + coef_b
            y = jnp.minimum(jnp.maximum(y, 1e-6), 1.0)
            return acc + y

        return lax.fori_loop(0, kdim // _L, vec_body, acc)

    return lax.fori_loop(0, _CH, row_body, acc)


def _chunk_transform(buf, a2, b2, lo2, hi2):
    kdim = buf.shape[1]

    def row_body(r, _):
        def vec_body(v, _):
            p = buf[r, pl.ds(v * _L, _L)]
            y = p * a2 + b2
            y = jnp.minimum(jnp.maximum(y, lo2), hi2)
            buf[r, pl.ds(v * _L, _L)] = y
            return 0

        return lax.fori_loop(0, kdim // _L, vec_body, 0)

    lax.fori_loop(0, _CH, row_body, 0)


def _hsum(vec, tmp_ref):
    idx = lax.iota(jnp.int32, _L)
    for sh in (8, 4, 2, 1):
        tmp_ref[...] = vec
        vec = vec + plsc.load_gather(tmp_ref, [jnp.bitwise_xor(idx, sh)])
    return vec


def _ue_coeffs(u_v, tmp_ref, kdim):
    ue = _hsum(u_v[...], tmp_ref)
    coef_a = 1.0 - ue
    coef_b = ue * (1.0 / kdim)
    return coef_a, coef_b


def _make_pass1(kdim, kdim2, rows_w):
    nch = rows_w // _CH

    def body(probs, x, u, partials, idx_v, u_v, buf0, buf1, acc_v,
             sem0, sem1):
        wid = _worker_id()
        base = wid * rows_w
        pltpu.sync_copy(x.at[pl.ds(base, rows_w)], idx_v)
        pltpu.sync_copy(u, u_v)
        coef_a, coef_b = _ue_coeffs(u_v, acc_v, kdim)

        def gather(ch, buf, sem):
            return pltpu.make_async_copy(
                probs.at[idx_v.at[pl.ds(ch * _CH, _CH)]], buf, sem
            )

        gather(0, buf0, sem0).start()
        gather(1, buf1, sem1).start()

        def group(g, acc):
            def half(ch, buf, sem, acc):
                gather(ch, buf, sem).wait()
                acc = _chunk_sum(buf, coef_a, coef_b, acc)

                @pl.when(ch + 2 < nch)
                def _():
                    gather(ch + 2, buf, sem).start()

                return acc

            acc = half(2 * g, buf0, sem0, acc)
            acc = half(2 * g + 1, buf1, sem1, acc)
            return acc

        acc = lax.fori_loop(0, nch // 2, group, jnp.zeros((_L,), jnp.float32))
        acc_v[...] = acc
        pltpu.sync_copy(acc_v, partials.at[wid])

    return pl.kernel(
        body,
        out_type=jax.ShapeDtypeStruct((_NW, _L), jnp.float32),
        mesh=plsc.VectorSubcoreMesh(core_axis_name="c", subcore_axis_name="s"),
        compiler_params=pltpu.CompilerParams(needs_layout_passes=False),
        scratch_types=[
            pltpu.VMEM((rows_w,), jnp.int32),
            pltpu.VMEM((_L,), jnp.float32),
            pltpu.VMEM((_CH, kdim2), jnp.float32),
            pltpu.VMEM((_CH, kdim2), jnp.float32),
            pltpu.VMEM((_L,), jnp.float32),
            pltpu.SemaphoreType.DMA,
            pltpu.SemaphoreType.DMA,
        ],
    )


def _make_pass2(kdim, kdim2, b2, rows_w):
    nch = rows_w // _CH
    ngrp = (nch + 2) // 3

    def body(probs, x, u, partials, out, idx_v, u_v, parts_v, tmp_v,
             bufs0, bufs1, bufs2, g0, g1, g2, w0, w1, w2):
        bufs = (bufs0, bufs1, bufs2)
        gsems = (g0, g1, g2)
        wsems = (w0, w1, w2)
        wid = _worker_id()
        base = wid * rows_w
        pltpu.sync_copy(x.at[pl.ds(base, rows_w)], idx_v)
        pltpu.sync_copy(u, u_v)
        pltpu.sync_copy(partials, parts_v)
        coef_a, coef_b = _ue_coeffs(u_v, tmp_v, kdim)

        def sum_partials(w, acc):
            return acc + parts_v[w, pl.ds(0, _L)]

        svec = lax.fori_loop(0, _NW, sum_partials,
                             jnp.zeros((_L,), jnp.float32))
        inv = 1.0 / _hsum(svec, tmp_v)
        a2 = coef_a * inv
        b2 = coef_b * inv
        lo2 = 1e-6 * inv
        hi2 = inv

        def gather(ch, buf, sem):
            return pltpu.make_async_copy(
                probs.at[idx_v.at[pl.ds(ch * _CH, _CH)]], buf, sem
            )

        def scatter(ch, buf, sem):
            return pltpu.make_async_copy(
                buf, out.at[pl.ds(base + ch * _CH, _CH)], sem
            )

        for l in range(3):
            gather(l, bufs[l], gsems[l]).start()

        def group(g, _):
            for l in range(3):
                ch = 3 * g + l

                @pl.when(ch < nch)
                def _():
                    gather(ch, bufs[l], gsems[l]).wait()
                    _chunk_transform(bufs[l], a2, b2, lo2, hi2)
                    scatter(ch, bufs[l], wsems[l]).start()

                    @pl.when(ch >= 3)
                    def _():
                        scatter(ch - 3, bufs[l], wsems[l]).wait()

                    @pl.when(ch + 3 < nch)
                    def _():
                        gather(ch + 3, bufs[l], gsems[l]).start()

            return 0

        lax.fori_loop(0, ngrp, group, 0)
        for ch in range(nch - 3, nch):
            scatter(ch, bufs[ch % 3], wsems[ch % 3]).wait()

    return pl.kernel(
        body,
        out_type=jax.ShapeDtypeStruct((b2, kdim2), jnp.float32),
        mesh=plsc.VectorSubcoreMesh(core_axis_name="c", subcore_axis_name="s"),
        compiler_params=pltpu.CompilerParams(needs_layout_passes=False),
        scratch_types=[
            pltpu.VMEM((rows_w,), jnp.int32),
            pltpu.VMEM((_L,), jnp.float32),
            pltpu.VMEM((_NW, _L), jnp.float32),
            pltpu.VMEM((_L,), jnp.float32),
            pltpu.VMEM((_CH, kdim2), jnp.float32),
            pltpu.VMEM((_CH, kdim2), jnp.float32),
            pltpu.VMEM((_CH, kdim2), jnp.float32),
            pltpu.SemaphoreType.DMA,
            pltpu.SemaphoreType.DMA,
            pltpu.SemaphoreType.DMA,
            pltpu.SemaphoreType.DMA,
            pltpu.SemaphoreType.DMA,
            pltpu.SemaphoreType.DMA,
        ],
    )


def kernel(probs, x, u, t_now, t_next):
    kdim = probs.shape[0]
    b = x.shape[0]
    kdim2 = kdim // 2
    probs2 = jnp.reshape(probs, (2 * kdim, kdim2))
    x_i32 = jnp.asarray(x).astype(jnp.int32)
    x2 = jnp.reshape(x_i32[:, None] * 2 + jnp.arange(2, dtype=jnp.int32),
                     (2 * b,))
    b2 = 2 * b
    rows_w = b2 // _NW
    u_vec = jnp.ravel(jnp.asarray(u)).astype(jnp.float32)
    u16 = jnp.zeros((_L,), jnp.float32).at[: u_vec.shape[0]].set(u_vec)

    partials = _make_pass1(kdim, kdim2, rows_w)(probs2, x2, u16)
    out2 = _make_pass2(kdim, kdim2, b2, rows_w)(probs2, x2, u16, partials)
    return jnp.reshape(out2, (b, kdim))


# speedup vs baseline: 1.9259x; 1.9259x over previous
import functools

import jax
import jax.numpy as jnp
from jax import lax
from jax.experimental import pallas as pl
from jax.experimental.pallas import tpu as pltpu
from jax.experimental.pallas import tpu_sc as plsc

_NC = 2
_NS = 16
_NW = _NC * _NS
_L = 16
_CH = 8


def _worker_id():
    return lax.axis_index("s") * _NC + lax.axis_index("c")


def _chunk_sum(buf, coef_a, coef_b, acc):
    kdim = buf.shape[1]
    vper = kdim // _L
    nvec = _CH * vper
    zero = jnp.zeros((_L,), jnp.float32)

    vlog = vper.bit_length() - 1

    @plsc.parallel_loop(0, nvec, step=2, unroll=8, carry=(acc, zero))
    def body(v, carry):
        a0, a1 = carry
        r = lax.shift_right_logical(v, vlog)
        c = lax.shift_left(jnp.bitwise_and(v, vper - 1), 4)
        p0 = buf[r, pl.ds(c, _L)]
        p1 = buf[r, pl.ds(c + _L, _L)]
        y0 = jnp.minimum(jnp.maximum(p0 * coef_a + coef_b, 1e-6), 1.0)
        y1 = jnp.minimum(jnp.maximum(p1 * coef_a + coef_b, 1e-6), 1.0)
        return (a0 + y0, a1 + y1)

    a0, a1 = body
    return a0 + a1


def _chunk_transform(buf, a2, b2, lo2, hi2):
    kdim = buf.shape[1]
    vper = kdim // _L
    nvec = _CH * vper

    vlog = vper.bit_length() - 1

    @plsc.parallel_loop(0, nvec, step=1, unroll=8)
    def body(v):
        r = lax.shift_right_logical(v, vlog)
        c = lax.shift_left(jnp.bitwise_and(v, vper - 1), 4)
        p = buf[r, pl.ds(c, _L)]
        y = jnp.minimum(jnp.maximum(p * a2 + b2, lo2), hi2)
        buf[r, pl.ds(c, _L)] = y


def _hsum(vec, tmp_ref):
    idx = lax.iota(jnp.int32, _L)
    for sh in (8, 4, 2, 1):
        tmp_ref[...] = vec
        vec = vec + plsc.load_gather(tmp_ref, [jnp.bitwise_xor(idx, sh)])
    return vec


def _ue_coeffs(u_v, tmp_ref, kdim):
    ue = _hsum(u_v[...], tmp_ref)
    coef_a = 1.0 - ue
    coef_b = ue * (1.0 / kdim)
    return coef_a, coef_b


def _make_pass1(kdim, kdim2, rows_w):
    nch = rows_w // _CH

    def body(probs, x, u, partials, idx_v, u_v, buf0, buf1, acc_v,
             sem0, sem1):
        wid = _worker_id()
        base = wid * rows_w
        pltpu.sync_copy(x.at[pl.ds(base, rows_w)], idx_v)
        pltpu.sync_copy(u, u_v)
        coef_a, coef_b = _ue_coeffs(u_v, acc_v, kdim)

        def gather(ch, buf, sem):
            return pltpu.make_async_copy(
                probs.at[idx_v.at[pl.ds(ch * _CH, _CH)]], buf, sem
            )

        gather(0, buf0, sem0).start()
        gather(1, buf1, sem1).start()

        def group(g, acc):
            def half(ch, buf, sem, acc):
                gather(ch, buf, sem).wait()
                acc = _chunk_sum(buf, coef_a, coef_b, acc)

                @pl.when(ch + 2 < nch)
                def _():
                    gather(ch + 2, buf, sem).start()

                return acc

            acc = half(2 * g, buf0, sem0, acc)
            acc = half(2 * g + 1, buf1, sem1, acc)
            return acc

        acc = lax.fori_loop(0, nch // 2, group, jnp.zeros((_L,), jnp.float32))
        acc_v[...] = acc
        pltpu.sync_copy(acc_v, partials.at[wid])

    return pl.kernel(
        body,
        out_type=jax.ShapeDtypeStruct((_NW, _L), jnp.float32),
        mesh=plsc.VectorSubcoreMesh(core_axis_name="c", subcore_axis_name="s"),
        compiler_params=pltpu.CompilerParams(needs_layout_passes=False),
        scratch_types=[
            pltpu.VMEM((rows_w,), jnp.int32),
            pltpu.VMEM((_L,), jnp.float32),
            pltpu.VMEM((_CH, kdim2), jnp.float32),
            pltpu.VMEM((_CH, kdim2), jnp.float32),
            pltpu.VMEM((_L,), jnp.float32),
            pltpu.SemaphoreType.DMA,
            pltpu.SemaphoreType.DMA,
        ],
    )


def _make_pass2(kdim, kdim2, b2, rows_w):
    nch = rows_w // _CH
    ngrp = (nch + 2) // 3

    def body(probs, x, u, partials, out, idx_v, u_v, parts_v, tmp_v,
             bufs0, bufs1, bufs2, g0, g1, g2, w0, w1, w2):
        bufs = (bufs0, bufs1, bufs2)
        gsems = (g0, g1, g2)
        wsems = (w0, w1, w2)
        wid = _worker_id()
        base = wid * rows_w
        pltpu.sync_copy(x.at[pl.ds(base, rows_w)], idx_v)
        pltpu.sync_copy(u, u_v)
        pltpu.sync_copy(partials, parts_v)
        coef_a, coef_b = _ue_coeffs(u_v, tmp_v, kdim)

        def sum_partials(w, acc):
            return acc + parts_v[w, pl.ds(0, _L)]

        svec = lax.fori_loop(0, _NW, sum_partials,
                             jnp.zeros((_L,), jnp.float32))
        inv = 1.0 / _hsum(svec, tmp_v)
        a2 = coef_a * inv
        b2 = coef_b * inv
        lo2 = 1e-6 * inv
        hi2 = inv

        def gather(ch, buf, sem):
            return pltpu.make_async_copy(
                probs.at[idx_v.at[pl.ds(ch * _CH, _CH)]], buf, sem
            )

        def scatter(ch, buf, sem):
            return pltpu.make_async_copy(
                buf, out.at[pl.ds(base + ch * _CH, _CH)], sem
            )

        for l in range(3):
            gather(l, bufs[l], gsems[l]).start()

        def group(g, _):
            for l in range(3):
                ch = 3 * g + l

                @pl.when(ch < nch)
                def _():
                    gather(ch, bufs[l], gsems[l]).wait()
                    _chunk_transform(bufs[l], a2, b2, lo2, hi2)
                    scatter(ch, bufs[l], wsems[l]).start()

                    @pl.when(ch >= 3)
                    def _():
                        scatter(ch - 3, bufs[l], wsems[l]).wait()

                    @pl.when(ch + 3 < nch)
                    def _():
                        gather(ch + 3, bufs[l], gsems[l]).start()

            return 0

        lax.fori_loop(0, ngrp, group, 0)
        for ch in range(nch - 3, nch):
            scatter(ch, bufs[ch % 3], wsems[ch % 3]).wait()

    return pl.kernel(
        body,
        out_type=jax.ShapeDtypeStruct((b2, kdim2), jnp.float32),
        mesh=plsc.VectorSubcoreMesh(core_axis_name="c", subcore_axis_name="s"),
        compiler_params=pltpu.CompilerParams(needs_layout_passes=False),
        scratch_types=[
            pltpu.VMEM((rows_w,), jnp.int32),
            pltpu.VMEM((_L,), jnp.float32),
            pltpu.VMEM((_NW, _L), jnp.float32),
            pltpu.VMEM((_L,), jnp.float32),
            pltpu.VMEM((_CH, kdim2), jnp.float32),
            pltpu.VMEM((_CH, kdim2), jnp.float32),
            pltpu.VMEM((_CH, kdim2), jnp.float32),
            pltpu.SemaphoreType.DMA,
            pltpu.SemaphoreType.DMA,
            pltpu.SemaphoreType.DMA,
            pltpu.SemaphoreType.DMA,
            pltpu.SemaphoreType.DMA,
            pltpu.SemaphoreType.DMA,
        ],
    )


def kernel(probs, x, u, t_now, t_next):
    kdim = probs.shape[0]
    b = x.shape[0]
    kdim2 = kdim // 2
    probs2 = jnp.reshape(probs, (2 * kdim, kdim2))
    x_i32 = jnp.asarray(x).astype(jnp.int32)
    x2 = jnp.reshape(x_i32[:, None] * 2 + jnp.arange(2, dtype=jnp.int32),
                     (2 * b,))
    b2 = 2 * b
    rows_w = b2 // _NW
    u_vec = jnp.ravel(jnp.asarray(u)).astype(jnp.float32)
    u16 = jnp.zeros((_L,), jnp.float32).at[: u_vec.shape[0]].set(u_vec)

    partials = _make_pass1(kdim, kdim2, rows_w)(probs2, x2, u16)
    out2 = _make_pass2(kdim, kdim2, b2, rows_w)(probs2, x2, u16, partials)
    return jnp.reshape(out2, (b, kdim))


# speedup vs baseline: 2.1193x; 1.1004x over previous
import jax
import jax.numpy as jnp
from jax.experimental import pallas as pl
from jax.experimental.pallas import tpu as pltpu

_R = 64


def _body(x_ref, u_ref, *refs):
    row_refs = refs[:_R]
    out_ref = refs[_R]
    acc_ref, s_ref = refs[_R + 1:]
    kdim = row_refs[0].shape[-1]
    kinv = 1.0 / kdim
    phase = pl.program_id(0)
    i = pl.program_id(1)
    ue = jnp.sum(u_ref[...])

    @pl.when(phase == 0)
    def _():
        @pl.when(i == 0)
        def _():
            acc_ref[...] = jnp.zeros_like(acc_ref)

        total = acc_ref[...]
        for j in range(_R):
            p = row_refs[j][0]
            p = p + ue * (kinv - p)
            p = jnp.clip(p, 1e-6, 1.0)
            total = total + p
        acc_ref[...] = total

    @pl.when(phase == 1)
    def _():
        @pl.when(i == 0)
        def _():
            s_ref[0] = 1.0 / jnp.sum(acc_ref[...])

        inv = s_ref[0]
        for j in range(_R):
            p = row_refs[j][0]
            p = p + ue * (kinv - p)
            p = jnp.clip(p, 1e-6, 1.0)
            out_ref[pl.ds(j, 1), :] = p * inv


def kernel(probs, x, u, t_now, t_next):
    kdim = probs.shape[0]
    b = x.shape[0]
    assert b % _R == 0
    x_i32 = jnp.asarray(x).astype(jnp.int32)
    u_vec = jnp.ravel(jnp.asarray(u)).astype(jnp.float32)
    u_pad = jnp.zeros((1, 128), jnp.float32).at[0, : u_vec.shape[0]].set(u_vec)

    probs3 = jnp.reshape(probs, (kdim, 1, kdim))

    def row_spec(j):
        return pl.BlockSpec(
            (1, 1, kdim), lambda ph, i, xr, j=j: (xr[i * _R + j], 0, 0)
        )

    grid_spec = pltpu.PrefetchScalarGridSpec(
        num_scalar_prefetch=1,
        grid=(2, b // _R),
        in_specs=[pl.BlockSpec((1, 128), lambda ph, i, xr: (0, 0))]
        + [row_spec(j) for j in range(_R)],
        out_specs=pl.BlockSpec(
            (_R, kdim), lambda ph, i, xr: (jnp.where(ph == 0, 0, i), 0)
        ),
        scratch_shapes=[
            pltpu.VMEM((1, kdim), jnp.float32),
            pltpu.SMEM((1,), jnp.float32),
        ],
    )

    return pl.pallas_call(
        _body,
        grid_spec=grid_spec,
        out_shape=jax.ShapeDtypeStruct((b, kdim), jnp.float32),
        compiler_params=pltpu.CompilerParams(
            dimension_semantics=("arbitrary", "arbitrary"),
        ),
    )(x_i32, u_pad, *([probs3] * _R))


# speedup vs baseline: 2.1582x; 1.0183x over previous
import jax
import jax.numpy as jnp
from jax.experimental import pallas as pl
from jax.experimental.pallas import tpu as pltpu

_R = 128


def _body(x_ref, u_ref, *refs):
    row_refs = refs[:_R]
    out_ref = refs[_R]
    acc_ref, s_ref = refs[_R + 1:]
    kdim = row_refs[0].shape[-1]
    kinv = 1.0 / kdim
    phase = pl.program_id(0)
    i = pl.program_id(1)
    ue = jnp.sum(u_ref[...])

    @pl.when(phase == 0)
    def _():
        @pl.when(i == 0)
        def _():
            acc_ref[...] = jnp.zeros_like(acc_ref)

        total = acc_ref[...]
        for j in range(_R):
            p = row_refs[j][0]
            p = p + ue * (kinv - p)
            p = jnp.clip(p, 1e-6, 1.0)
            total = total + p
        acc_ref[...] = total

    @pl.when(phase == 1)
    def _():
        @pl.when(i == 0)
        def _():
            s_ref[0] = 1.0 / jnp.sum(acc_ref[...])

        inv = s_ref[0]
        for j in range(_R):
            p = row_refs[j][0]
            p = p + ue * (kinv - p)
            p = jnp.clip(p, 1e-6, 1.0)
            out_ref[pl.ds(j, 1), :] = p * inv


def kernel(probs, x, u, t_now, t_next):
    kdim = probs.shape[0]
    b = x.shape[0]
    assert b % _R == 0
    x_i32 = jnp.asarray(x).astype(jnp.int32)
    u_vec = jnp.ravel(jnp.asarray(u)).astype(jnp.float32)
    u_pad = jnp.zeros((1, 128), jnp.float32).at[0, : u_vec.shape[0]].set(u_vec)

    probs3 = jnp.reshape(probs, (kdim, 1, kdim))

    def row_spec(j):
        return pl.BlockSpec(
            (1, 1, kdim), lambda ph, i, xr, j=j: (xr[i * _R + j], 0, 0)
        )

    grid_spec = pltpu.PrefetchScalarGridSpec(
        num_scalar_prefetch=1,
        grid=(2, b // _R),
        in_specs=[pl.BlockSpec((1, 128), lambda ph, i, xr: (0, 0))]
        + [row_spec(j) for j in range(_R)],
        out_specs=pl.BlockSpec(
            (_R, kdim), lambda ph, i, xr: (jnp.where(ph == 0, 0, i), 0)
        ),
        scratch_shapes=[
            pltpu.VMEM((1, kdim), jnp.float32),
            pltpu.SMEM((1,), jnp.float32),
        ],
    )

    return pl.pallas_call(
        _body,
        grid_spec=grid_spec,
        out_shape=jax.ShapeDtypeStruct((b, kdim), jnp.float32),
        compiler_params=pltpu.CompilerParams(
            dimension_semantics=("arbitrary", "arbitrary"),
        ),
    )(x_i32, u_pad, *([probs3] * _R))
